# baseline (speedup 1.0000x reference)
import functools

import jax
import jax.numpy as jnp
from jax import lax
from jax.experimental import pallas as pl
from jax.experimental.pallas import tpu as pltpu
from jax.experimental.pallas import tpu_sc as plsc

VOCAB = 100000
Z_DIM = 32
BATCH = 1024
TILE_V = 2048
NV = (VOCAB + TILE_V - 1) // TILE_V


def _gather_sc(table, idx):
    info = plsc.get_sparse_core_info()
    nc, ns = info.num_cores, info.num_subcores
    nw = nc * ns
    bpw = BATCH // nw
    mesh = plsc.VectorSubcoreMesh(core_axis_name="c", subcore_axis_name="s")

    @functools.partial(
        pl.kernel,
        mesh=mesh,
        out_type=jax.ShapeDtypeStruct((BATCH, Z_DIM), jnp.float32),
        scratch_types=[
            pltpu.VMEM((bpw,), jnp.int32),
            pltpu.VMEM((bpw, Z_DIM), jnp.float32),
            pltpu.SemaphoreType.DMA,
        ],
        compiler_params=pltpu.CompilerParams(use_tc_tiling_on_sc=False),
    )
    def gather(table_hbm, idx_hbm, out_hbm, idx_v, rows_v, sem):
        wid = lax.axis_index("s") * nc + lax.axis_index("c")
        base = wid * bpw
        pltpu.sync_copy(idx_hbm.at[pl.ds(base, bpw)], idx_v)
        pltpu.async_copy(table_hbm.at[idx_v], rows_v, sem).wait()
        pltpu.sync_copy(rows_v, out_hbm.at[pl.ds(base, bpw)])

    return gather(table, idx)


def _fused_body(emb_ref, w_ref, b_ref, out_ref, m_ref, s_ref):
    p = pl.program_id(0)
    j = pl.program_id(1)

    @pl.when((p == 0) & (j == 0))
    def _init():
        m_ref[...] = jnp.full((BATCH, 1), -jnp.inf, jnp.float32)
        s_ref[...] = jnp.zeros((BATCH, 1), jnp.float32)

    scores = lax.dot_general(
        emb_ref[...], w_ref[...], (((1,), (1,)), ((), ())),
        preferred_element_type=jnp.float32,
    ) + b_ref[...]
    col = j * TILE_V + lax.broadcasted_iota(jnp.int32, (BATCH, TILE_V), 1)
    scores = jnp.where(col < VOCAB, scores, -jnp.inf)

    @pl.when(p == 0)
    def _stats():
        m_old = m_ref[...]
        m_new = jnp.maximum(m_old, jnp.max(scores, axis=1, keepdims=True))
        s_ref[...] = s_ref[...] * jnp.exp(m_old - m_new) + jnp.sum(
            jnp.exp(scores - m_new), axis=1, keepdims=True)
        m_ref[...] = m_new

    @pl.when(p == 1)
    def _write():
        out_ref[...] = scores - m_ref[...] - jnp.log(s_ref[...])


def _fused_logsoftmax(emb, w, b2):
    return pl.pallas_call(
        _fused_body,
        grid=(2, NV),
        in_specs=[
            pl.BlockSpec((BATCH, Z_DIM), lambda p, j: (0, 0)),
            pl.BlockSpec((TILE_V, Z_DIM), lambda p, j: (j, 0)),
            pl.BlockSpec((1, TILE_V), lambda p, j: (0, j)),
        ],
        out_specs=pl.BlockSpec((BATCH, TILE_V), lambda p, j: (0, j * p)),
        out_shape=jax.ShapeDtypeStruct((BATCH, VOCAB), jnp.float32),
        scratch_shapes=[
            pltpu.VMEM((BATCH, 1), jnp.float32),
            pltpu.VMEM((BATCH, 1), jnp.float32),
        ],
    )(emb, w, b2)


def kernel(input_word, emb_table, W_out, b_out):
    idx = input_word.astype(jnp.int32)
    emb = _gather_sc(emb_table, idx)
    return _fused_logsoftmax(emb, W_out, b_out.reshape(1, VOCAB))


# speedup vs baseline: 1.0158x; 1.0158x over previous
import functools

import jax
import jax.numpy as jnp
from jax import lax
from jax.experimental import pallas as pl
from jax.experimental.pallas import tpu as pltpu
from jax.experimental.pallas import tpu_sc as plsc

VOCAB = 100000
Z_DIM = 32
BATCH = 1024
TILE_V = 2048
NV = (VOCAB + TILE_V - 1) // TILE_V


def _gather_sc(table, idx):
    info = plsc.get_sparse_core_info()
    nc, ns = info.num_cores, info.num_subcores
    nw = nc * ns
    bpw = BATCH // nw
    mesh = plsc.VectorSubcoreMesh(core_axis_name="c", subcore_axis_name="s")

    @functools.partial(
        pl.kernel,
        mesh=mesh,
        out_type=jax.ShapeDtypeStruct((BATCH, Z_DIM), jnp.float32),
        scratch_types=[
            pltpu.VMEM((bpw,), jnp.int32),
            pltpu.VMEM((bpw, Z_DIM), jnp.float32),
            pltpu.SemaphoreType.DMA,
        ],
        compiler_params=pltpu.CompilerParams(use_tc_tiling_on_sc=False),
    )
    def gather(table_hbm, idx_hbm, out_hbm, idx_v, rows_v, sem):
        wid = lax.axis_index("s") * nc + lax.axis_index("c")
        base = wid * bpw
        pltpu.sync_copy(idx_hbm.at[pl.ds(base, bpw)], idx_v)
        pltpu.async_copy(table_hbm.at[idx_v], rows_v, sem).wait()
        pltpu.sync_copy(rows_v, out_hbm.at[pl.ds(base, bpw)])

    return gather(table, idx)


def _fused_body(emb_ref, w_ref, b_ref, out_ref, m_ref, s_ref):
    p = pl.program_id(0)
    j = pl.program_id(1)

    @pl.when((p == 0) & (j == 0))
    def _init():
        m_ref[...] = jnp.full((BATCH, 1), -jnp.inf, jnp.float32)
        s_ref[...] = jnp.zeros((BATCH, 1), jnp.float32)

    scores = lax.dot_general(
        emb_ref[...], w_ref[...], (((1,), (1,)), ((), ())),
        preferred_element_type=jnp.float32,
    ) + b_ref[...]

    def _stats(sc):
        m_old = m_ref[...]
        m_new = jnp.maximum(m_old, jnp.max(sc, axis=1, keepdims=True))
        s_new = s_ref[...] * jnp.exp(m_old - m_new) + jnp.sum(
            jnp.exp(sc - m_new), axis=1, keepdims=True)
        s_ref[...] = s_new
        m_ref[...] = m_new
        return m_new, s_new

    @pl.when((p == 0) & (j < NV - 1))
    def _stats_full():
        _stats(scores)

    @pl.when((p == 0) & (j == NV - 1))
    def _stats_masked():
        col = j * TILE_V + lax.broadcasted_iota(jnp.int32, (BATCH, TILE_V), 1)
        m_new, s_new = _stats(jnp.where(col < VOCAB, scores, -jnp.inf))
        m_ref[...] = m_new + jnp.log(s_new)

    @pl.when(p == 1)
    def _write():
        out_ref[...] = scores - m_ref[...]


def _fused_logsoftmax(emb, w, b2):
    return pl.pallas_call(
        _fused_body,
        grid=(2, NV),
        in_specs=[
            pl.BlockSpec((BATCH, Z_DIM), lambda p, j: (0, 0)),
            pl.BlockSpec((TILE_V, Z_DIM), lambda p, j: (j, 0)),
            pl.BlockSpec((1, TILE_V), lambda p, j: (0, j)),
        ],
        out_specs=pl.BlockSpec((BATCH, TILE_V), lambda p, j: (0, j * p)),
        out_shape=jax.ShapeDtypeStruct((BATCH, VOCAB), jnp.float32),
        scratch_shapes=[
            pltpu.VMEM((BATCH, 1), jnp.float32),
            pltpu.VMEM((BATCH, 1), jnp.float32),
        ],
    )(emb, w, b2)


def kernel(input_word, emb_table, W_out, b_out):
    idx = input_word.astype(jnp.int32)
    emb = _gather_sc(emb_table, idx)
    return _fused_logsoftmax(
        emb.astype(jnp.bfloat16), W_out.astype(jnp.bfloat16),
        b_out.reshape(1, VOCAB))
